# Initial kernel scaffold; baseline (speedup 1.0000x reference)
import jax
import jax.numpy as jnp
from jax.experimental import pallas as pl


def kernel(x, edge_index, edge_attr, pos, batch, W_in, g_in, b_in, W_e, g_e, b_e, W_p, g_p, b_p, W_convs, eps, g_n, b_n, W_o1, g_o, b_o, W_o2):
    raise NotImplementedError("write your pallas kernel here")


# speedup vs baseline: 2.9031x; 2.9031x over previous
import functools
import jax
import jax.numpy as jnp
from jax import lax
from jax.experimental import pallas as pl
from jax.experimental.pallas import tpu as pltpu
from jax.experimental.pallas import tpu_sc as plsc

N, E, DF, DE, H, OUT, L, G = 10000, 320000, 128, 16, 128, 128, 3, 128

CH = 128
NW = 32
EP = ((E + CH * NW - 1) // (CH * NW)) * (CH * NW)
CPW = EP // (CH * NW)
NPAD = 10240
RPT = NPAD // 16
F32 = jnp.float32

_sc_mesh = plsc.VectorSubcoreMesh(core_axis_name="c", subcore_axis_name="s")
_sc_params = pltpu.CompilerParams(needs_layout_passes=False)


def _dis_body(pos_hbm, src_hbm, dst_hbm, o0, o1, o2, posv, ia, ib, d0, d1, d2):
    c = lax.axis_index("c")
    s = lax.axis_index("s")
    wid = s * 2 + c
    pltpu.sync_copy(pos_hbm, posv)
    dbufs = (d0, d1, d2)

    def chunk(i, carry):
        base = (wid * CPW + i) * CH
        pltpu.sync_copy(src_hbm.at[pl.ds(base, CH)], ia)
        pltpu.sync_copy(dst_hbm.at[pl.ds(base, CH)], ib)

        def grp(g, carry2):
            sl = pl.ds(g * 16, 16)
            sa = ia[sl] * 8
            sb = ib[sl] * 8
            for comp in range(3):
                va = plsc.load_gather(posv, [sa + comp])
                vb = plsc.load_gather(posv, [sb + comp])
                dbufs[comp][sl] = jnp.abs(va - vb)
            return carry2

        lax.fori_loop(0, CH // 16, grp, 0)
        pltpu.sync_copy(d0, o0.at[pl.ds(base, CH)])
        pltpu.sync_copy(d1, o1.at[pl.ds(base, CH)])
        pltpu.sync_copy(d2, o2.at[pl.ds(base, CH)])
        return carry

    lax.fori_loop(0, CPW, chunk, 0)


@jax.jit
def _sc_dis(pos8, src_p, dst_p):
    k = pl.kernel(
        _dis_body,
        mesh=_sc_mesh,
        compiler_params=_sc_params,
        out_type=[jax.ShapeDtypeStruct((EP,), F32)] * 3,
        scratch_types=[
            pltpu.VMEM((N * 8,), F32),
            pltpu.VMEM((CH,), jnp.int32),
            pltpu.VMEM((CH,), jnp.int32),
            pltpu.VMEM((CH,), F32),
            pltpu.VMEM((CH,), F32),
            pltpu.VMEM((CH,), F32),
        ],
    )
    return k(pos8, src_p, dst_p)


def _edge_body(h_hbm, ea_hbm, src_hbm, dst_hbm, out_hbm, agg, ea_buf, h_buf,
               isrc, idst):
    c = lax.axis_index("c")
    s = lax.axis_index("s")
    wid = s * 2 + c

    def zrow(i, carry):
        for j in range(H // 16):
            h_buf[i, pl.ds(j * 16, 16)] = jnp.zeros((16,), F32)
        return carry

    lax.fori_loop(0, CH, zrow, 0)
    for k in range(RPT // CH):
        pltpu.sync_copy(h_buf, agg.at[pl.ds(s * RPT + k * CH, CH)])
    plsc.subcore_barrier()

    def chunk(i, carry):
        base = (wid * CPW + i) * CH
        pltpu.sync_copy(src_hbm.at[pl.ds(base, CH)], isrc)
        pltpu.sync_copy(dst_hbm.at[pl.ds(base, CH)], idst)
        pltpu.sync_copy(ea_hbm.at[pl.ds(base, CH)], ea_buf)
        pltpu.sync_copy(h_hbm.at[isrc], h_buf)

        def row(e, carry2):
            for j in range(H // 16):
                sl = pl.ds(j * 16, 16)
                ea_buf[e, sl] = jnp.maximum(ea_buf[e, sl] + h_buf[e, sl], 0.0)
            return carry2

        lax.fori_loop(0, CH, row, 0)
        pltpu.sync_copy(ea_buf, agg.at[idst], add=True)
        return carry

    lax.fori_loop(0, CPW, chunk, 0)
    plsc.subcore_barrier()

    for k in range(RPT // CH):
        r0 = s * RPT + k * CH
        pltpu.sync_copy(agg.at[pl.ds(r0, CH)], h_buf)
        pltpu.sync_copy(h_buf, out_hbm.at[pl.ds(c * NPAD + r0, CH)])


@jax.jit
def _sc_edge_pass(h, ea_enc, src_p, dst_p):
    k = pl.kernel(
        _edge_body,
        mesh=_sc_mesh,
        compiler_params=_sc_params,
        out_type=jax.ShapeDtypeStruct((2 * NPAD, H), F32),
        scratch_types=[
            pltpu.VMEM_SHARED((NPAD, H), F32),
            pltpu.VMEM((CH, H), F32),
            pltpu.VMEM((CH, H), F32),
            pltpu.VMEM((CH,), jnp.int32),
            pltpu.VMEM((CH,), jnp.int32),
        ],
    )
    return k(h, ea_enc, src_p, dst_p)


def _h0_body(x_ref, w_ref, g_ref, b_ref, o_ref):
    z = jnp.dot(x_ref[...], w_ref[...], preferred_element_type=F32)
    mu = jnp.mean(z, axis=0, keepdims=True)
    var = jnp.mean(z * z, axis=0, keepdims=True) - mu * mu
    zn = (z - mu) * lax.rsqrt(var + 1e-5) * g_ref[...] + b_ref[...]
    o_ref[...] = jnp.maximum(zn, 0.0)


@jax.jit
def _tc_h0(x, W_in, g2, b2):
    return pl.pallas_call(
        _h0_body,
        out_shape=jax.ShapeDtypeStruct((N, H), F32),
    )(x, W_in, g2, b2)


_STB = 2048


_DN0 = (((0,), (0,)), ((), ()))


def _stats_body(ea_ref, d0_ref, d1_ref, d2_ref, we_ref, wp_ref,
                sze_ref, qze_ref, szd_ref, qzd_ref):
    @pl.when(pl.program_id(0) == 0)
    def _():
        sze_ref[...] = jnp.zeros_like(sze_ref)
        qze_ref[...] = jnp.zeros_like(qze_ref)
        szd_ref[...] = jnp.zeros_like(szd_ref)
        qzd_ref[...] = jnp.zeros_like(qzd_ref)

    ze = jnp.dot(ea_ref[...], we_ref[...], preferred_element_type=F32)
    d = jnp.concatenate([d0_ref[...], d1_ref[...], d2_ref[...]], axis=0)
    zd = lax.dot_general(d, wp_ref[...], _DN0, preferred_element_type=F32)
    sze_ref[...] += jnp.sum(ze, axis=0, keepdims=True)
    qze_ref[...] += jnp.sum(ze * ze, axis=0, keepdims=True)
    szd_ref[...] += jnp.sum(zd, axis=0, keepdims=True)
    qzd_ref[...] += jnp.sum(zd * zd, axis=0, keepdims=True)


@jax.jit
def _tc_stats(ea_p, d0, d1, d2, W_e, W_p):
    nb = EP // _STB
    drow = pl.BlockSpec((1, _STB), lambda i: (0, i))
    srow = pl.BlockSpec((1, H), lambda i: (0, 0))
    return pl.pallas_call(
        _stats_body,
        grid=(nb,),
        in_specs=[
            pl.BlockSpec((_STB, DE), lambda i: (i, 0)), drow, drow, drow,
            pl.BlockSpec((DE, H), lambda i: (0, 0)),
            pl.BlockSpec((3, H), lambda i: (0, 0)),
        ],
        out_specs=[srow, srow, srow, srow],
        out_shape=[jax.ShapeDtypeStruct((1, H), F32)] * 4,
    )(ea_p, d0, d1, d2, W_e, W_p)


def _enc_body(ea_ref, d0_ref, d1_ref, d2_ref, sze_ref, qze_ref, szd_ref,
              qzd_ref, we_ref, ge_ref, be_ref, wp_ref, gp_ref, bp_ref, o_ref):
    inv_e = 1.0 / E

    me = sze_ref[...] * inv_e
    ve = qze_ref[...] * inv_e - me * me
    sce = ge_ref[...] * lax.rsqrt(ve + 1e-5)
    a = jnp.dot(ea_ref[...], we_ref[...], preferred_element_type=F32)
    a = jnp.maximum((a - me) * sce + be_ref[...], 0.0)

    md = szd_ref[...] * inv_e
    vd = qzd_ref[...] * inv_e - md * md
    scd = gp_ref[...] * lax.rsqrt(vd + 1e-5)
    d = jnp.concatenate([d0_ref[...], d1_ref[...], d2_ref[...]], axis=0)
    b = lax.dot_general(d, wp_ref[...], _DN0, preferred_element_type=F32)
    b = jnp.maximum((b - md) * scd + bp_ref[...], 0.0)
    o_ref[...] = a + b


@jax.jit
def _tc_encode(ea_p, d0, d1, d2, sze, qze, szd, qzd, W_e, g_e2, b_e2, W_p,
               g_p2, b_p2):
    nb = EP // _STB
    drow = pl.BlockSpec((1, _STB), lambda i: (0, i))
    srow = pl.BlockSpec((1, H), lambda i: (0, 0))
    small = [
        srow, srow, srow, srow,
        pl.BlockSpec((DE, H), lambda i: (0, 0)),
        srow, srow,
        pl.BlockSpec((3, H), lambda i: (0, 0)),
        srow, srow,
    ]
    return pl.pallas_call(
        _enc_body,
        grid=(nb,),
        in_specs=[
            pl.BlockSpec((_STB, DE), lambda i: (i, 0)), drow, drow, drow,
        ] + small,
        out_specs=pl.BlockSpec((_STB, H), lambda i: (i, 0)),
        out_shape=jax.ShapeDtypeStruct((EP, H), F32),
    )(ea_p, d0, d1, d2, sze, qze, szd, qzd, W_e, g_e2, b_e2, W_p, g_p2, b_p2)


def _layer_body(h_ref, a0_ref, a1_ref, w_ref, eps_ref, g_ref, b_ref, o_ref,
                *, residual):
    h = h_ref[...]
    u = (1.0 + eps_ref[0, 0]) * h + a0_ref[...] + a1_ref[...]
    z = jnp.dot(u, w_ref[...], preferred_element_type=F32)
    mu = jnp.mean(z, axis=0, keepdims=True)
    var = jnp.mean(z * z, axis=0, keepdims=True) - mu * mu
    zn = (z - mu) * lax.rsqrt(var + 1e-5) * g_ref[...] + b_ref[...]
    r = jnp.maximum(zn, 0.0)
    if residual:
        r = r + h
    o_ref[...] = r


@functools.partial(jax.jit, static_argnames=("residual",))
def _tc_layer(h, a0, a1, W, eps1, g2, b2, residual):
    return pl.pallas_call(
        functools.partial(_layer_body, residual=residual),
        out_shape=jax.ShapeDtypeStruct((N, H), F32),
    )(h, a0, a1, W, eps1, g2, b2)


def _pool_body(h_ref, b_ref, w1_ref, g_ref, bo_ref, w2_ref, o_ref):
    oh = (b_ref[...] == lax.broadcasted_iota(jnp.int32, (1, G), 1)).astype(F32)
    dn = (((0,), (0,)), ((), ()))
    gp = lax.dot_general(oh, h_ref[...], dn, preferred_element_type=F32,
                         precision=lax.Precision.HIGHEST)
    q = jnp.dot(gp, w1_ref[...], preferred_element_type=F32)
    mu = jnp.mean(q, axis=0, keepdims=True)
    var = jnp.mean(q * q, axis=0, keepdims=True) - mu * mu
    qn = (q - mu) * lax.rsqrt(var + 1e-5) * g_ref[...] + bo_ref[...]
    o_ref[...] = jnp.dot(jnp.maximum(qn, 0.0), w2_ref[...],
                         preferred_element_type=F32)


@jax.jit
def _tc_pool(h, b2, W_o1, g2, bo2, W_o2):
    return pl.pallas_call(
        _pool_body,
        out_shape=jax.ShapeDtypeStruct((G, OUT), F32),
    )(h, b2, W_o1, g2, bo2, W_o2)


def kernel(x, edge_index, edge_attr, pos, batch, W_in, g_in, b_in, W_e, g_e,
           b_e, W_p, g_p, b_p, W_convs, eps, g_n, b_n, W_o1, g_o, b_o, W_o2):
    src = edge_index[0].astype(jnp.int32)
    dst = edge_index[1].astype(jnp.int32)
    padn = EP - E
    src_p = jnp.concatenate([src, jnp.zeros((padn,), jnp.int32)])
    dst_dis = jnp.concatenate([dst, jnp.zeros((padn,), jnp.int32)])
    dst_conv = jnp.concatenate([dst, jnp.full((padn,), N, jnp.int32)])
    pos8 = jnp.pad(pos.astype(F32), ((0, 0), (0, 8 - pos.shape[1]))).reshape(-1)
    ea_p = jnp.pad(edge_attr, ((0, padn), (0, 0)))

    r2 = lambda a: a.reshape(1, -1)

    d0, d1, d2 = _sc_dis(pos8, src_p, dst_dis)
    d0, d1, d2 = r2(d0), r2(d1), r2(d2)
    sze, qze, szd, qzd = _tc_stats(ea_p, d0, d1, d2, W_e, W_p)
    ea_enc = _tc_encode(ea_p, d0, d1, d2, sze, qze, szd, qzd,
                        W_e, r2(g_e), r2(b_e), W_p, r2(g_p), r2(b_p))
    h = _tc_h0(x, W_in, r2(g_in), r2(b_in))

    for l in range(L):
        agg = _sc_edge_pass(h, ea_enc, src_p, dst_conv)
        h = _tc_layer(h, agg[:N], agg[NPAD:NPAD + N], W_convs[l],
                      eps[l].reshape(1, 1), r2(g_n[l]), r2(b_n[l]),
                      residual=(l > 0))

    return _tc_pool(h, batch.astype(jnp.int32).reshape(N, 1),
                    W_o1, r2(g_o), r2(b_o), W_o2)
